# Initial kernel scaffold; baseline (speedup 1.0000x reference)
import jax
import jax.numpy as jnp
from jax.experimental import pallas as pl


def kernel(x, edge_index_r0, edge_index_r1, W_T0, W_T1, W_A0, W_A1, ln_gamma, ln_beta):
    raise NotImplementedError("write your pallas kernel here")


# speedup vs baseline: 12.0175x; 12.0175x over previous
import functools

import jax
import jax.numpy as jnp
from jax import lax
from jax.experimental import pallas as pl
from jax.experimental.pallas import tpu as pltpu
from jax.experimental.pallas import tpu_sc as plsc

L = 16
NC = 2
NS = 16
NW = NC * NS
CH = 128


def _lane():
    return jnp.arange(L, dtype=jnp.int32)


def _tc_transform(x, W_T0, W_T1, wa0_row, wa1_row):
    N, d = x.shape
    BR = 400

    def body(x_ref, wt0_ref, wt1_ref, wa0_ref, wa1_ref, g0_ref, g1_ref):
        xb = x_ref[...]
        g0_ref[...] = jnp.dot(xb, wt0_ref[...],
                              preferred_element_type=jnp.float32) * wa0_ref[...]
        g1_ref[...] = jnp.dot(xb, wt1_ref[...],
                              preferred_element_type=jnp.float32) * wa1_ref[...]

    return pl.pallas_call(
        body,
        grid=(N // BR,),
        in_specs=[
            pl.BlockSpec((BR, d), lambda i: (i, 0)),
            pl.BlockSpec((d, d), lambda i: (0, 0)),
            pl.BlockSpec((d, d), lambda i: (0, 0)),
            pl.BlockSpec((1, d), lambda i: (0, 0)),
            pl.BlockSpec((1, d), lambda i: (0, 0)),
        ],
        out_specs=[pl.BlockSpec((BR, d), lambda i: (i, 0))] * 2,
        out_shape=[jax.ShapeDtypeStruct((N, d), jnp.float32)] * 2,
    )(x, W_T0, W_T1, wa0_row, wa1_row)


def _att_merge(att_parts):
    npad = att_parts.shape[1]

    def body(a_ref, o_ref):
        o_ref[...] = 1.0 / jnp.sum(a_ref[...], axis=0, keepdims=True)

    return pl.pallas_call(
        body, out_shape=jax.ShapeDtypeStruct((1, npad), jnp.float32)
    )(att_parts)


def _make_sc_pass_a(N, d, E, NPAD):
    EPW = E // NW
    NCHF = EPW // CH
    NFULL = NCHF * CH
    TAIL = EPW - NFULL
    EPAD = NFULL + 2 * L
    NPAIR = NCHF // 2
    assert NCHF % 2 == 1
    mesh = plsc.VectorSubcoreMesh(core_axis_name="c", subcore_axis_name="s",
                                  num_cores=NC, num_subcores=NS)
    nk = d // L

    @functools.partial(
        pl.kernel,
        out_type=[jax.ShapeDtypeStruct((E,), jnp.float32),
                  jax.ShapeDtypeStruct((E,), jnp.float32),
                  jax.ShapeDtypeStruct((NW * NPAD,), jnp.float32)],
        mesh=mesh,
        compiler_params=pltpu.CompilerParams(needs_layout_passes=False),
        scratch_types=[
            pltpu.VMEM((EPAD,), jnp.int32),
            pltpu.VMEM((EPAD,), jnp.int32),
            pltpu.VMEM((CH, 128), jnp.float32),
            pltpu.VMEM((CH, 128), jnp.float32),
            pltpu.VMEM((CH, 128), jnp.float32),
            pltpu.VMEM((CH, 128), jnp.float32),
            pltpu.VMEM((CH,), jnp.float32),
            pltpu.VMEM((NPAD,), jnp.float32),
            pltpu.SemaphoreType.DMA,
            pltpu.SemaphoreType.DMA,
        ],
    )
    def pass_a(g0, g1, x, src0, dst0, src1, dst1, s0_out, s1_out, att_out,
               sia, dia, gr0, gr1, xr0, xr1, sbuf, attp, sg0, sg1):
        cid = lax.axis_index("c")
        sid = lax.axis_index("s")
        wid = cid * NS + sid

        def zb(i, _):
            attp[pl.ds(i * L, L)] = jnp.zeros((L,), jnp.float32)
            return 0
        lax.fori_loop(0, NPAD // L, zb, 0)

        def start_gather(g, ci, grows, xrows, sem):
            pltpu.async_copy(g.at[sia.at[pl.ds(ci * CH, CH)]], grows, sem)
            pltpu.async_copy(x.at[dia.at[pl.ds(ci * CH, CH)]], xrows, sem)

        def wait_gather(g, grows, xrows, sem):
            pltpu.make_async_copy(g.at[sia.at[pl.ds(0, CH)]], grows, sem).wait()
            pltpu.make_async_copy(x.at[dia.at[pl.ds(0, CH)]], xrows, sem).wait()

        def dots(n, grows, xrows):
            @plsc.parallel_loop(0, n, 1, unroll=2)
            def edot(e):
                acc = grows[e, pl.ds(0, L)] * xrows[e, pl.ds(0, L)]
                for k in range(1, nk):
                    acc = acc + grows[e, pl.ds(k * L, L)] * xrows[e, pl.ds(k * L, L)]
                cum = plsc.cumsum(acc)
                plsc.store_scatter(sbuf, [jnp.full((L,), e, jnp.int32)], cum,
                                   mask=_lane() == L - 1)

        def exp_and_att(j, off, valid):
            s16 = jnp.exp(sbuf[pl.ds(j * L, L)])
            sbuf[pl.ds(j * L, L)] = s16
            d16 = dia[pl.ds(off + j * L, L)]
            plsc.addupdate_scatter(attp, [d16], s16, mask=_lane() < valid)

        def compute_chunk(ci, grows, xrows, s_out):
            dots(CH, grows, xrows)

            def jb(j, _):
                exp_and_att(j, ci * CH, L)
                return 0
            lax.fori_loop(0, CH // L, jb, 0)
            pltpu.sync_copy(sbuf, s_out.at[pl.ds(wid * EPW + ci * CH, CH)])

        for src_e, dst_e, g, s_out in ((src0, dst0, g0, s0_out),
                                       (src1, dst1, g1, s1_out)):
            dia[pl.ds(NFULL, L)] = jnp.zeros((L,), jnp.int32)
            dia[pl.ds(NFULL + L, L)] = jnp.zeros((L,), jnp.int32)
            pltpu.sync_copy(src_e.at[pl.ds(wid * EPW, EPW)],
                            sia.at[pl.ds(0, EPW)])
            pltpu.sync_copy(dst_e.at[pl.ds(wid * EPW, EPW)],
                            dia.at[pl.ds(0, EPW)])
            start_gather(g, 0, gr0, xr0, sg0)

            def pair(i, _, g=g, s_out=s_out):
                ci0 = 2 * i
                start_gather(g, ci0 + 1, gr1, xr1, sg1)
                wait_gather(g, gr0, xr0, sg0)
                compute_chunk(ci0, gr0, xr0, s_out)
                start_gather(g, ci0 + 2, gr0, xr0, sg0)
                wait_gather(g, gr1, xr1, sg1)
                compute_chunk(ci0 + 1, gr1, xr1, s_out)
                return 0
            lax.fori_loop(0, NPAIR, pair, 0)

            wait_gather(g, gr0, xr0, sg0)
            compute_chunk(NCHF - 1, gr0, xr0, s_out)

            if TAIL:
                pltpu.async_copy(g.at[sia.at[pl.ds(NFULL, TAIL)]],
                                 gr0.at[pl.ds(0, TAIL)], sg0).wait()
                pltpu.async_copy(x.at[dia.at[pl.ds(NFULL, TAIL)]],
                                 xr0.at[pl.ds(0, TAIL)], sg0).wait()
                dots(TAIL, gr0, xr0)
                exp_and_att(0, NFULL, TAIL)
                pltpu.sync_copy(sbuf.at[pl.ds(0, TAIL)],
                                s_out.at[pl.ds(wid * EPW + NFULL, TAIL)])

        pltpu.sync_copy(attp, att_out.at[pl.ds(wid * NPAD, NPAD)])

    return pass_a


def _make_sc_pass_b(N, d, E, NPAD):
    CB = 64
    EPW = E // NW
    NCHB = EPW // CB
    NFULL = NCHB * CB
    TAIL = EPW - NFULL
    EPAD = NFULL + 2 * L
    NPAIR = NCHB // 2
    assert NCHB % 2 == 0
    ROWS_PT = NPAD // NS
    mesh = plsc.VectorSubcoreMesh(core_axis_name="c", subcore_axis_name="s",
                                  num_cores=NC, num_subcores=NS)
    nk = d // L

    @functools.partial(
        pl.kernel,
        out_type=jax.ShapeDtypeStruct((NC, NPAD, d), jnp.float32),
        mesh=mesh,
        compiler_params=pltpu.CompilerParams(needs_layout_passes=False),
        scratch_types=[
            pltpu.VMEM((EPAD,), jnp.int32),
            pltpu.VMEM((EPAD,), jnp.int32),
            pltpu.VMEM((CB,), jnp.float32),
            pltpu.VMEM((CB,), jnp.float32),
            pltpu.VMEM((CB, 128), jnp.float32),
            pltpu.VMEM((CB, 128), jnp.float32),
            pltpu.VMEM((CB, 128), jnp.float32),
            pltpu.VMEM((NPAD,), jnp.float32),
            pltpu.VMEM_SHARED((NPAD, 128), jnp.float32),
            pltpu.SemaphoreType.DMA,
            pltpu.SemaphoreType.DMA,
        ],
    )
    def pass_b(x, src0, dst0, src1, dst1, s0, s1, inv_att, h_out,
               sia, dia, sbuf, abuf, xr0, xr1, scaled, invav, hacc, sg0, sg1):
        cid = lax.axis_index("c")
        sid = lax.axis_index("s")
        wid = cid * NS + sid

        pltpu.sync_copy(inv_att, invav)

        def zrow(e, _):
            for k in range(nk):
                scaled[e, pl.ds(k * L, L)] = jnp.zeros((L,), jnp.float32)
            return 0
        lax.fori_loop(0, CB, zrow, 0)
        for c in range(ROWS_PT // CB):
            pltpu.sync_copy(scaled, hacc.at[pl.ds(sid * ROWS_PT + c * CB, CB)])
        plsc.subcore_barrier()

        def start_gather(ci, xrows, sem):
            pltpu.async_copy(x.at[sia.at[pl.ds(ci * CB, CB)]], xrows, sem)

        def wait_gather(xrows, sem):
            pltpu.make_async_copy(x.at[sia.at[pl.ds(0, CB)]], xrows, sem).wait()

        def scale_rows(n, xrows):
            @plsc.parallel_loop(0, n, 1, unroll=2)
            def erow(e):
                a16 = plsc.load_gather(abuf, [jnp.full((L,), e, jnp.int32)])
                for k in range(nk):
                    scaled[e, pl.ds(k * L, L)] = xrows[e, pl.ds(k * L, L)] * a16

        def compute_chunk(ci, xrows, s_in):
            base = ci * CB
            pltpu.sync_copy(s_in.at[pl.ds(wid * EPW + base, CB)], sbuf)

            def ab(j, _):
                d16 = dia[pl.ds(base + j * L, L)]
                ia16 = plsc.load_gather(invav, [d16])
                abuf[pl.ds(j * L, L)] = sbuf[pl.ds(j * L, L)] * ia16
                return 0
            lax.fori_loop(0, CB // L, ab, 0)
            scale_rows(CB, xrows)

            def jb(j, _):
                d16 = dia[pl.ds(base + j * L, L)]
                pltpu.sync_copy(scaled.at[pl.ds(j * L, L)], hacc.at[d16],
                                add=True)
                return 0
            lax.fori_loop(0, CB // L, jb, 0)

        for src_e, dst_e, s_in in ((src0, dst0, s0), (src1, dst1, s1)):
            dia[pl.ds(NFULL, L)] = jnp.zeros((L,), jnp.int32)
            dia[pl.ds(NFULL + L, L)] = jnp.zeros((L,), jnp.int32)
            pltpu.sync_copy(src_e.at[pl.ds(wid * EPW, EPW)],
                            sia.at[pl.ds(0, EPW)])
            pltpu.sync_copy(dst_e.at[pl.ds(wid * EPW, EPW)],
                            dia.at[pl.ds(0, EPW)])

            start_gather(0, xr0, sg0)

            def pair(i, _, s_in=s_in):
                ci0 = 2 * i
                start_gather(ci0 + 1, xr1, sg1)
                wait_gather(xr0, sg0)
                compute_chunk(ci0, xr0, s_in)
                start_gather(ci0 + 2, xr0, sg0)
                wait_gather(xr1, sg1)
                compute_chunk(ci0 + 1, xr1, s_in)
                return 0
            lax.fori_loop(0, NPAIR - 1, pair, 0)

            start_gather(NCHB - 1, xr1, sg1)
            wait_gather(xr0, sg0)
            compute_chunk(NCHB - 2, xr0, s_in)
            wait_gather(xr1, sg1)
            compute_chunk(NCHB - 1, xr1, s_in)

            if TAIL:
                pltpu.async_copy(x.at[sia.at[pl.ds(NFULL, TAIL)]],
                                 xr0.at[pl.ds(0, TAIL)], sg0).wait()
                sbuf[pl.ds(0, L)] = jnp.zeros((L,), jnp.float32)
                pltpu.sync_copy(s_in.at[pl.ds(wid * EPW + NFULL, TAIL)],
                                sbuf.at[pl.ds(0, TAIL)])
                d16 = dia[pl.ds(NFULL, L)]
                ia16 = plsc.load_gather(invav, [d16])
                abuf[pl.ds(0, L)] = sbuf[pl.ds(0, L)] * ia16
                scale_rows(TAIL, xr0)
                for e in range(TAIL, L):
                    for k in range(nk):
                        scaled[e, pl.ds(k * L, L)] = jnp.zeros((L,), jnp.float32)
                pltpu.sync_copy(scaled.at[pl.ds(0, L)], hacc.at[d16], add=True)

        plsc.subcore_barrier()
        pltpu.sync_copy(hacc.at[pl.ds(sid * ROWS_PT, ROWS_PT)],
                        h_out.at[cid, pl.ds(sid * ROWS_PT, ROWS_PT)])

    return pass_b


def _tc_final(h0, h1, gamma_row, beta_row):
    N, d = h0.shape
    BR = 400

    def body(h0_ref, h1_ref, g_ref, b_ref, o_ref):
        h = jnp.maximum(h0_ref[...] + h1_ref[...], 0.0)
        mu = jnp.mean(h, axis=1, keepdims=True)
        c = h - mu
        var = jnp.mean(c * c, axis=1, keepdims=True)
        o_ref[...] = c * lax.rsqrt(var + 1e-5) * g_ref[...] + b_ref[...]

    return pl.pallas_call(
        body,
        grid=(N // BR,),
        in_specs=[
            pl.BlockSpec((BR, d), lambda i: (i, 0)),
            pl.BlockSpec((BR, d), lambda i: (i, 0)),
            pl.BlockSpec((1, d), lambda i: (0, 0)),
            pl.BlockSpec((1, d), lambda i: (0, 0)),
        ],
        out_specs=pl.BlockSpec((BR, d), lambda i: (i, 0)),
        out_shape=jax.ShapeDtypeStruct((N, d), jnp.float32),
    )(h0, h1, gamma_row, beta_row)


def kernel(x, edge_index_r0, edge_index_r1, W_T0, W_T1, W_A0, W_A1,
           ln_gamma, ln_beta):
    N, d = x.shape
    E = edge_index_r0.shape[1]
    NPAD = ((N + NW * L - 1) // (NW * L)) * (NW * L)

    src0, dst0 = edge_index_r0[0], edge_index_r0[1]
    src1, dst1 = edge_index_r1[0], edge_index_r1[1]
    g0, g1 = _tc_transform(x, W_T0, W_T1,
                           W_A0.reshape(1, d), W_A1.reshape(1, d))
    s0, s1, att_parts = _make_sc_pass_a(N, d, E, NPAD)(
        g0, g1, x, src0, dst0, src1, dst1)
    inv_att = _att_merge(att_parts.reshape(NW, NPAD)).reshape(NPAD)
    h_parts = _make_sc_pass_b(N, d, E, NPAD)(
        x, src0, dst0, src1, dst1, s0, s1, inv_att)
    return _tc_final(h_parts[0, :N], h_parts[1, :N],
                     ln_gamma.reshape(1, d), ln_beta.reshape(1, d))


# speedup vs baseline: 14.2413x; 1.1851x over previous
import functools

import jax
import jax.numpy as jnp
from jax import lax
from jax.experimental import pallas as pl
from jax.experimental.pallas import tpu as pltpu
from jax.experimental.pallas import tpu_sc as plsc

L = 16
NC = 2
NS = 16
NW = NC * NS
CH = 128


def _lane():
    return jnp.arange(L, dtype=jnp.int32)


def _tc_transform(x, W_T0, W_T1, wa0_row, wa1_row):
    N, d = x.shape
    BR = 400

    def body(x_ref, wt0_ref, wt1_ref, wa0_ref, wa1_ref, g0_ref, g1_ref):
        xb = x_ref[...]
        g0_ref[...] = jnp.dot(xb, wt0_ref[...],
                              preferred_element_type=jnp.float32) * wa0_ref[...]
        g1_ref[...] = jnp.dot(xb, wt1_ref[...],
                              preferred_element_type=jnp.float32) * wa1_ref[...]

    return pl.pallas_call(
        body,
        grid=(N // BR,),
        in_specs=[
            pl.BlockSpec((BR, d), lambda i: (i, 0)),
            pl.BlockSpec((d, d), lambda i: (0, 0)),
            pl.BlockSpec((d, d), lambda i: (0, 0)),
            pl.BlockSpec((1, d), lambda i: (0, 0)),
            pl.BlockSpec((1, d), lambda i: (0, 0)),
        ],
        out_specs=[pl.BlockSpec((BR, d), lambda i: (i, 0))] * 2,
        out_shape=[jax.ShapeDtypeStruct((N, d), jnp.float32)] * 2,
    )(x, W_T0, W_T1, wa0_row, wa1_row)


def _att_merge(att_parts):
    npad = att_parts.shape[1]

    def body(a_ref, o_ref):
        o_ref[...] = 1.0 / jnp.sum(a_ref[...], axis=0, keepdims=True)

    return pl.pallas_call(
        body, out_shape=jax.ShapeDtypeStruct((1, npad), jnp.float32)
    )(att_parts)


def _make_sc_pass_a(N, d, E, NPAD):
    EPW = E // NW
    NCHF = EPW // CH
    NFULL = NCHF * CH
    TAIL = EPW - NFULL
    EPAD = NFULL + 2 * L
    NPAIR = NCHF // 2
    assert NCHF % 2 == 1
    mesh = plsc.VectorSubcoreMesh(core_axis_name="c", subcore_axis_name="s",
                                  num_cores=NC, num_subcores=NS)
    nk = d // L

    @functools.partial(
        pl.kernel,
        out_type=[jax.ShapeDtypeStruct((E,), jnp.float32),
                  jax.ShapeDtypeStruct((E,), jnp.float32),
                  jax.ShapeDtypeStruct((NW * NPAD,), jnp.float32)],
        mesh=mesh,
        compiler_params=pltpu.CompilerParams(needs_layout_passes=False),
        scratch_types=[
            pltpu.VMEM((EPAD,), jnp.int32),
            pltpu.VMEM((EPAD,), jnp.int32),
            pltpu.VMEM((CH, 128), jnp.float32),
            pltpu.VMEM((CH, 128), jnp.float32),
            pltpu.VMEM((CH, 128), jnp.float32),
            pltpu.VMEM((CH, 128), jnp.float32),
            pltpu.VMEM((CH,), jnp.float32),
            pltpu.VMEM((CH,), jnp.float32),
            pltpu.VMEM((NPAD,), jnp.float32),
            pltpu.SemaphoreType.DMA,
            pltpu.SemaphoreType.DMA,
            pltpu.SemaphoreType.DMA,
        ],
    )
    def pass_a(g0, g1, x, src0, dst0, src1, dst1, s0_out, s1_out, att_out,
               sia, dia, gr0, gr1, xr0, xr1, sb0, sb1, attp, sg0, sg1, st):
        cid = lax.axis_index("c")
        sid = lax.axis_index("s")
        wid = cid * NS + sid

        def zb(i, _):
            attp[pl.ds(i * L, L)] = jnp.zeros((L,), jnp.float32)
            return 0
        lax.fori_loop(0, NPAD // L, zb, 0)

        def start_gather(g, ci, grows, xrows, sem):
            pltpu.async_copy(g.at[sia.at[pl.ds(ci * CH, CH)]], grows, sem)
            pltpu.async_copy(x.at[dia.at[pl.ds(ci * CH, CH)]], xrows, sem)

        def wait_gather(g, grows, xrows, sem):
            pltpu.make_async_copy(g.at[sia.at[pl.ds(0, CH)]], grows, sem).wait()
            pltpu.make_async_copy(x.at[dia.at[pl.ds(0, CH)]], xrows, sem).wait()

        def dots(n, grows, xrows, sbuf):
            @plsc.parallel_loop(0, n, 1, unroll=2)
            def edot(e):
                acc = grows[e, pl.ds(0, L)] * xrows[e, pl.ds(0, L)]
                for k in range(1, nk):
                    acc = acc + grows[e, pl.ds(k * L, L)] * xrows[e, pl.ds(k * L, L)]
                cum = plsc.cumsum(acc)
                plsc.store_scatter(sbuf, [jnp.full((L,), e, jnp.int32)], cum,
                                   mask=_lane() == L - 1)

        def exp_and_att(j, off, valid, sbuf):
            s16 = jnp.exp(sbuf[pl.ds(j * L, L)])
            sbuf[pl.ds(j * L, L)] = s16
            d16 = dia[pl.ds(off + j * L, L)]
            plsc.addupdate_scatter(attp, [d16], s16, mask=_lane() < valid)

        def compute_chunk(ci, grows, xrows, s_out, sbuf, sync_store):
            dots(CH, grows, xrows, sbuf)

            def jb(j, _):
                exp_and_att(j, ci * CH, L, sbuf)
                return 0
            lax.fori_loop(0, CH // L, jb, 0)
            dst = s_out.at[pl.ds(wid * EPW + ci * CH, CH)]
            if sync_store:
                pltpu.sync_copy(sbuf, dst)
            else:
                pltpu.async_copy(sbuf, dst, st)

        def drain_store(s_out, sbuf):
            pltpu.make_async_copy(sbuf, s_out.at[pl.ds(0, CH)], st).wait()

        for src_e, dst_e, g, s_out in ((src0, dst0, g0, s0_out),
                                       (src1, dst1, g1, s1_out)):
            dia[pl.ds(NFULL, L)] = jnp.zeros((L,), jnp.int32)
            dia[pl.ds(NFULL + L, L)] = jnp.zeros((L,), jnp.int32)
            pltpu.sync_copy(src_e.at[pl.ds(wid * EPW, EPW)],
                            sia.at[pl.ds(0, EPW)])
            pltpu.sync_copy(dst_e.at[pl.ds(wid * EPW, EPW)],
                            dia.at[pl.ds(0, EPW)])
            start_gather(g, 0, gr0, xr0, sg0)

            def pair(i, _, g=g, s_out=s_out):
                ci0 = 2 * i
                start_gather(g, ci0 + 1, gr1, xr1, sg1)
                wait_gather(g, gr0, xr0, sg0)
                compute_chunk(ci0, gr0, xr0, s_out, sb0, False)
                start_gather(g, ci0 + 2, gr0, xr0, sg0)
                wait_gather(g, gr1, xr1, sg1)
                compute_chunk(ci0 + 1, gr1, xr1, s_out, sb1, False)
                drain_store(s_out, sb0)
                drain_store(s_out, sb1)
                return 0
            lax.fori_loop(0, NPAIR, pair, 0)

            wait_gather(g, gr0, xr0, sg0)
            compute_chunk(NCHF - 1, gr0, xr0, s_out, sb0, True)

            if TAIL:
                pltpu.async_copy(g.at[sia.at[pl.ds(NFULL, TAIL)]],
                                 gr0.at[pl.ds(0, TAIL)], sg0).wait()
                pltpu.async_copy(x.at[dia.at[pl.ds(NFULL, TAIL)]],
                                 xr0.at[pl.ds(0, TAIL)], sg0).wait()
                dots(TAIL, gr0, xr0, sb0)
                exp_and_att(0, NFULL, TAIL, sb0)
                pltpu.sync_copy(sb0.at[pl.ds(0, TAIL)],
                                s_out.at[pl.ds(wid * EPW + NFULL, TAIL)])

        pltpu.sync_copy(attp, att_out.at[pl.ds(wid * NPAD, NPAD)])

    return pass_a


def _make_sc_pass_b(N, d, E, NPAD):
    CB = 64
    EPW = E // NW
    NCHB = EPW // CB
    NFULL = NCHB * CB
    TAIL = EPW - NFULL
    EPAD = NFULL + 2 * L
    NPAIR = NCHB // 2
    assert NCHB % 2 == 0
    ROWS_PT = NPAD // NS
    mesh = plsc.VectorSubcoreMesh(core_axis_name="c", subcore_axis_name="s",
                                  num_cores=NC, num_subcores=NS)
    nk = d // L
    NG = CB // L

    @functools.partial(
        pl.kernel,
        out_type=jax.ShapeDtypeStruct((NC, NPAD, d), jnp.float32),
        mesh=mesh,
        compiler_params=pltpu.CompilerParams(needs_layout_passes=False),
        scratch_types=[
            pltpu.VMEM((EPAD,), jnp.int32),
            pltpu.VMEM((EPAD,), jnp.int32),
            pltpu.VMEM((CB,), jnp.float32),
            pltpu.VMEM((CB,), jnp.float32),
            pltpu.VMEM((CB,), jnp.float32),
            pltpu.VMEM((CB,), jnp.float32),
            pltpu.VMEM((CB,), jnp.float32),
            pltpu.VMEM((CB,), jnp.float32),
            pltpu.VMEM((CB, 128), jnp.float32),
            pltpu.VMEM((CB, 128), jnp.float32),
            pltpu.VMEM((CB, 128), jnp.float32),
            pltpu.VMEM((CB, 128), jnp.float32),
            pltpu.VMEM_SHARED((NPAD, 128), jnp.float32),
            pltpu.SemaphoreType.DMA,
            pltpu.SemaphoreType.DMA,
            pltpu.SemaphoreType.DMA,
        ],
    )
    def pass_b(x, src0, dst0, src1, dst1, s0, s1, inv_att, h_out,
               sia, dia, sb0, sb1, ib0, ib1, ab0, ab1, xr0, xr1, sc0, sc1,
               hacc, sg0, sg1, ss):
        cid = lax.axis_index("c")
        sid = lax.axis_index("s")
        wid = cid * NS + sid

        def zrow(e, _):
            for k in range(nk):
                sc0[e, pl.ds(k * L, L)] = jnp.zeros((L,), jnp.float32)
            return 0
        lax.fori_loop(0, CB, zrow, 0)
        for c in range(ROWS_PT // CB):
            pltpu.sync_copy(sc0, hacc.at[pl.ds(sid * ROWS_PT + c * CB, CB)])
        plsc.subcore_barrier()

        def start_pf(ci, xrows, sbuf, ibuf, sem, s_in):
            pltpu.async_copy(x.at[sia.at[pl.ds(ci * CB, CB)]], xrows, sem)
            pltpu.async_copy(s_in.at[pl.ds(wid * EPW + ci * CB, CB)], sbuf, sem)
            pltpu.async_copy(inv_att.at[dia.at[pl.ds(ci * CB, CB)]], ibuf, sem)

        def wait_pf(xrows, sbuf, ibuf, sem, s_in):
            pltpu.make_async_copy(x.at[sia.at[pl.ds(0, CB)]], xrows, sem).wait()
            pltpu.make_async_copy(s_in.at[pl.ds(0, CB)], sbuf, sem).wait()
            pltpu.make_async_copy(inv_att.at[dia.at[pl.ds(0, CB)]], ibuf,
                                  sem).wait()

        def scale_rows(n, xrows, abuf, scaled):
            @plsc.parallel_loop(0, n, 1, unroll=2)
            def erow(e):
                a16 = plsc.load_gather(abuf, [jnp.full((L,), e, jnp.int32)])
                for k in range(nk):
                    scaled[e, pl.ds(k * L, L)] = xrows[e, pl.ds(k * L, L)] * a16

        def compute_chunk(ci, xrows, sbuf, ibuf, abuf, scaled):
            def ajb(j, _):
                abuf[pl.ds(j * L, L)] = (sbuf[pl.ds(j * L, L)]
                                         * ibuf[pl.ds(j * L, L)])
                return 0
            lax.fori_loop(0, NG, ajb, 0)
            scale_rows(CB, xrows, abuf, scaled)

            def jb(j, _):
                d16 = dia[pl.ds(ci * CB + j * L, L)]
                pltpu.async_copy(scaled.at[pl.ds(j * L, L)], hacc.at[d16], ss,
                                 add=True)
                return 0
            lax.fori_loop(0, NG, jb, 0)

        def drain_scatter(scaled):
            def jd(j, _):
                d16 = dia[pl.ds(j * L, L)]
                pltpu.make_async_copy(scaled.at[pl.ds(j * L, L)], hacc.at[d16],
                                      ss).wait()
                return 0
            lax.fori_loop(0, NG, jd, 0)

        for src_e, dst_e, s_in in ((src0, dst0, s0), (src1, dst1, s1)):
            dia[pl.ds(NFULL, L)] = jnp.zeros((L,), jnp.int32)
            dia[pl.ds(NFULL + L, L)] = jnp.zeros((L,), jnp.int32)
            pltpu.sync_copy(src_e.at[pl.ds(wid * EPW, EPW)],
                            sia.at[pl.ds(0, EPW)])
            pltpu.sync_copy(dst_e.at[pl.ds(wid * EPW, EPW)],
                            dia.at[pl.ds(0, EPW)])

            start_pf(0, xr0, sb0, ib0, sg0, s_in)

            def pair(i, _, s_in=s_in):
                ci0 = 2 * i
                start_pf(ci0 + 1, xr1, sb1, ib1, sg1, s_in)
                wait_pf(xr0, sb0, ib0, sg0, s_in)
                compute_chunk(ci0, xr0, sb0, ib0, ab0, sc0)
                start_pf(ci0 + 2, xr0, sb0, ib0, sg0, s_in)
                wait_pf(xr1, sb1, ib1, sg1, s_in)
                compute_chunk(ci0 + 1, xr1, sb1, ib1, ab1, sc1)
                drain_scatter(sc0)
                drain_scatter(sc1)
                return 0
            lax.fori_loop(0, NPAIR - 1, pair, 0)

            start_pf(NCHB - 1, xr1, sb1, ib1, sg1, s_in)
            wait_pf(xr0, sb0, ib0, sg0, s_in)
            compute_chunk(NCHB - 2, xr0, sb0, ib0, ab0, sc0)
            wait_pf(xr1, sb1, ib1, sg1, s_in)
            compute_chunk(NCHB - 1, xr1, sb1, ib1, ab1, sc1)
            drain_scatter(sc0)
            drain_scatter(sc1)

            if TAIL:
                pltpu.async_copy(x.at[sia.at[pl.ds(NFULL, TAIL)]],
                                 xr0.at[pl.ds(0, TAIL)], sg0).wait()
                sb0[pl.ds(0, L)] = jnp.zeros((L,), jnp.float32)
                pltpu.sync_copy(s_in.at[pl.ds(wid * EPW + NFULL, TAIL)],
                                sb0.at[pl.ds(0, TAIL)])
                pltpu.async_copy(inv_att.at[dia.at[pl.ds(NFULL, L)]],
                                 ib0.at[pl.ds(0, L)], sg0).wait()
                ab0[pl.ds(0, L)] = sb0[pl.ds(0, L)] * ib0[pl.ds(0, L)]
                scale_rows(TAIL, xr0, ab0, sc0)
                for e in range(TAIL, L):
                    for k in range(nk):
                        sc0[e, pl.ds(k * L, L)] = jnp.zeros((L,), jnp.float32)
                d16 = dia[pl.ds(NFULL, L)]
                pltpu.sync_copy(sc0.at[pl.ds(0, L)], hacc.at[d16], add=True)

        plsc.subcore_barrier()
        pltpu.sync_copy(hacc.at[pl.ds(sid * ROWS_PT, ROWS_PT)],
                        h_out.at[cid, pl.ds(sid * ROWS_PT, ROWS_PT)])

    return pass_b


def _tc_final(h0, h1, gamma_row, beta_row):
    N, d = h0.shape
    BR = 400

    def body(h0_ref, h1_ref, g_ref, b_ref, o_ref):
        h = jnp.maximum(h0_ref[...] + h1_ref[...], 0.0)
        mu = jnp.mean(h, axis=1, keepdims=True)
        c = h - mu
        var = jnp.mean(c * c, axis=1, keepdims=True)
        o_ref[...] = c * lax.rsqrt(var + 1e-5) * g_ref[...] + b_ref[...]

    return pl.pallas_call(
        body,
        grid=(N // BR,),
        in_specs=[
            pl.BlockSpec((BR, d), lambda i: (i, 0)),
            pl.BlockSpec((BR, d), lambda i: (i, 0)),
            pl.BlockSpec((1, d), lambda i: (0, 0)),
            pl.BlockSpec((1, d), lambda i: (0, 0)),
        ],
        out_specs=pl.BlockSpec((BR, d), lambda i: (i, 0)),
        out_shape=jax.ShapeDtypeStruct((N, d), jnp.float32),
    )(h0, h1, gamma_row, beta_row)


def kernel(x, edge_index_r0, edge_index_r1, W_T0, W_T1, W_A0, W_A1,
           ln_gamma, ln_beta):
    N, d = x.shape
    E = edge_index_r0.shape[1]
    NPAD = ((N + NW * L - 1) // (NW * L)) * (NW * L)

    src0, dst0 = edge_index_r0[0], edge_index_r0[1]
    src1, dst1 = edge_index_r1[0], edge_index_r1[1]
    g0, g1 = _tc_transform(x, W_T0, W_T1,
                           W_A0.reshape(1, d), W_A1.reshape(1, d))
    s0, s1, att_parts = _make_sc_pass_a(N, d, E, NPAD)(
        g0, g1, x, src0, dst0, src1, dst1)
    inv_att = _att_merge(att_parts.reshape(NW, NPAD)).reshape(NPAD)
    h_parts = _make_sc_pass_b(N, d, E, NPAD)(
        x, src0, dst0, src1, dst1, s0, s1, inv_att)
    return _tc_final(h_parts[0, :N], h_parts[1, :N],
                     ln_gamma.reshape(1, d), ln_beta.reshape(1, d))
